# baseline (speedup 1.0000x reference)
import functools

import jax
import jax.numpy as jnp
from jax.experimental import pallas as pl
from jax.experimental.pallas import tpu as pltpu

_BLK = 400
_CACHE = 5


def _s1_kernel(x_ref, W1_ref, s1_ref):
    s1_ref[...] = jnp.dot(x_ref[...], W1_ref[...],
                          preferred_element_type=jnp.float32
                          ).astype(jnp.bfloat16)


def _fused_kernel(s1_ref, adj_ref, W4_ref, b1_ref, b4_ref, out_ref,
                  bufs_ref, cache_ref, s4_ref, sems, *, blk, nb, cache):
    p = pl.program_id(0)
    i = pl.program_id(1)
    s = p * nb + i

    def _copy(ordinal, block):
        slot = jax.lax.rem(ordinal, 2)
        return pltpu.make_async_copy(
            adj_ref.at[pl.ds(block * blk, blk), :],
            bufs_ref.at[slot],
            sems.at[slot],
        )

    @pl.when(s == 0)
    def _first():
        _copy(0, 0).start()

    ns = s + 1
    np_ = ns // nb
    ni = ns - np_ * nb
    nxt_needs = jnp.logical_and(ns < 2 * nb,
                                jnp.logical_or(np_ == 0, ni < nb - cache))
    nxt_block = jnp.where(np_ == 0, ni, ni + cache)
    nxt_ord = jnp.where(np_ == 0, ni, nb + ni)

    @pl.when(nxt_needs)
    def _prefetch():
        _copy(nxt_ord, nxt_block).start()

    cur_needs = jnp.logical_or(p == 0, i < nb - cache)
    cur_ord = jnp.where(p == 0, i, nb + i)
    cur_block = jnp.where(p == 0, i, i + cache)

    @pl.when(cur_needs)
    def _wait():
        _copy(cur_ord, cur_block).wait()

    slot = jax.lax.rem(cur_ord, 2)

    def _finish(o):
        m = jnp.max(o, axis=1, keepdims=True)
        lse = jnp.log(jnp.sum(jnp.exp(o - m), axis=1, keepdims=True)) + m
        out_ref[...] = o - lse

    @pl.when(p == 0)
    def _phase0():
        a = bufs_ref[slot]
        h = jnp.dot(a, s1_ref[...].astype(jnp.float32),
                    preferred_element_type=jnp.float32)
        h = jnp.maximum(h + b1_ref[...], 0.0)
        s4_ref[pl.ds(i * blk, blk), :] = jnp.dot(
            h, W4_ref[...], preferred_element_type=jnp.float32
        ).astype(jnp.bfloat16)

        @pl.when(i < cache)
        def _stash():
            cache_ref[i] = bufs_ref[slot].astype(jnp.float8_e4m3fn)

    @pl.when(jnp.logical_and(p == 1, i < nb - cache))
    def _phase1_hbm():
        _finish(jnp.dot(bufs_ref[slot], s4_ref[...].astype(jnp.float32),
                        preferred_element_type=jnp.float32) + b4_ref[...])

    @pl.when(jnp.logical_and(p == 1, i >= nb - cache))
    def _phase1_cached():
        q = cache_ref[i - (nb - cache)].astype(jnp.bfloat16)
        _finish(jnp.dot(q, s4_ref[...],
                        preferred_element_type=jnp.float32) + b4_ref[...])


def kernel(x, adj, W1, b1, W4, b4):
    n, nfeat = x.shape
    nhid = W1.shape[1]
    nclass = W4.shape[1]

    b1_2d = b1.reshape(1, nhid)
    b4_2d = b4.reshape(1, nclass)

    blk = _BLK if n % _BLK == 0 else n
    nb = n // blk
    cache = _CACHE if nb > _CACHE else 0

    s1 = pl.pallas_call(
        _s1_kernel,
        in_specs=[
            pl.BlockSpec((n, nfeat), lambda: (0, 0)),
            pl.BlockSpec((nfeat, nhid), lambda: (0, 0)),
        ],
        out_specs=pl.BlockSpec((n, nhid), lambda: (0, 0)),
        out_shape=jax.ShapeDtypeStruct((n, nhid), jnp.bfloat16),
    )(x, W1)

    body = functools.partial(_fused_kernel, blk=blk, nb=nb, cache=cache)
    out = pl.pallas_call(
        body,
        grid=(2, nb),
        in_specs=[
            pl.BlockSpec((n, nhid), lambda p, i: (0, 0)),
            pl.BlockSpec(memory_space=pl.ANY),
            pl.BlockSpec((nhid, nclass), lambda p, i: (0, 0)),
            pl.BlockSpec((1, nhid), lambda p, i: (0, 0)),
            pl.BlockSpec((1, nclass), lambda p, i: (0, 0)),
        ],
        out_specs=pl.BlockSpec(
            (blk, nclass),
            lambda p, i: (jnp.where(p == 0, i, (i + cache) % nb), 0)),
        out_shape=jax.ShapeDtypeStruct((n, nclass), jnp.float32),
        scratch_shapes=[
            pltpu.VMEM((2, blk, n), jnp.float32),
            pltpu.VMEM((max(cache, 1), blk, n), jnp.float8_e4m3fn),
            pltpu.VMEM((n, nclass), jnp.bfloat16),
            pltpu.SemaphoreType.DMA((2,)),
        ],
        compiler_params=pltpu.CompilerParams(
            dimension_semantics=("arbitrary", "arbitrary"),
        ),
    )(s1, adj, W4, b1_2d, b4_2d)
    return out


# speedup vs baseline: 1.0019x; 1.0019x over previous
import functools

import jax
import jax.numpy as jnp
from jax.experimental import pallas as pl
from jax.experimental.pallas import tpu as pltpu

_BLK = 400
_CACHE = 4


def _s1_kernel(x_ref, W1_ref, s1_ref):
    s1_ref[...] = jnp.dot(x_ref[...], W1_ref[...],
                          preferred_element_type=jnp.float32
                          ).astype(jnp.bfloat16)


def _fused_kernel(s1_ref, adj_ref, W4_ref, b1_ref, b4_ref, out_ref,
                  bufs_ref, cache_ref, s4_ref, sems, *, blk, nb, cache):
    p = pl.program_id(0)
    i = pl.program_id(1)
    s = p * nb + i

    def _copy(ordinal, block):
        slot = jax.lax.rem(ordinal, 2)
        return pltpu.make_async_copy(
            adj_ref.at[pl.ds(block * blk, blk), :],
            bufs_ref.at[slot],
            sems.at[slot],
        )

    @pl.when(s == 0)
    def _first():
        _copy(0, 0).start()

    ns = s + 1
    np_ = ns // nb
    ni = ns - np_ * nb
    nxt_needs = jnp.logical_and(ns < 2 * nb,
                                jnp.logical_or(np_ == 0, ni < nb - cache))
    nxt_block = jnp.where(np_ == 0, ni, ni + cache)
    nxt_ord = jnp.where(np_ == 0, ni, nb + ni)

    @pl.when(nxt_needs)
    def _prefetch():
        _copy(nxt_ord, nxt_block).start()

    cur_needs = jnp.logical_or(p == 0, i < nb - cache)
    cur_ord = jnp.where(p == 0, i, nb + i)
    cur_block = jnp.where(p == 0, i, i + cache)

    @pl.when(cur_needs)
    def _wait():
        _copy(cur_ord, cur_block).wait()

    slot = jax.lax.rem(cur_ord, 2)

    def _finish(o):
        m = jnp.max(o, axis=1, keepdims=True)
        lse = jnp.log(jnp.sum(jnp.exp(o - m), axis=1, keepdims=True)) + m
        out_ref[...] = o - lse

    @pl.when(p == 0)
    def _phase0():
        a = bufs_ref[slot]
        h = jnp.dot(a, s1_ref[...].astype(jnp.float32),
                    preferred_element_type=jnp.float32)
        h = jnp.maximum(h + b1_ref[...], 0.0)
        s4_ref[pl.ds(i * blk, blk), :] = jnp.dot(
            h, W4_ref[...], preferred_element_type=jnp.float32
        ).astype(jnp.bfloat16)

        @pl.when(i < cache)
        def _stash():
            cache_ref[i] = bufs_ref[slot].astype(jnp.float8_e4m3fn)

    @pl.when(jnp.logical_and(p == 1, i < nb - cache))
    def _phase1_hbm():
        _finish(jnp.dot(bufs_ref[slot], s4_ref[...].astype(jnp.float32),
                        preferred_element_type=jnp.float32) + b4_ref[...])

    @pl.when(jnp.logical_and(p == 1, i >= nb - cache))
    def _phase1_cached():
        q = cache_ref[i - (nb - cache)].astype(jnp.bfloat16)
        _finish(jnp.dot(q, s4_ref[...],
                        preferred_element_type=jnp.float32) + b4_ref[...])


def kernel(x, adj, W1, b1, W4, b4):
    n, nfeat = x.shape
    nhid = W1.shape[1]
    nclass = W4.shape[1]

    b1_2d = b1.reshape(1, nhid)
    b4_2d = b4.reshape(1, nclass)

    blk = _BLK if n % _BLK == 0 else n
    nb = n // blk
    cache = _CACHE if nb > _CACHE else 0

    s1 = pl.pallas_call(
        _s1_kernel,
        in_specs=[
            pl.BlockSpec((n, nfeat), lambda: (0, 0)),
            pl.BlockSpec((nfeat, nhid), lambda: (0, 0)),
        ],
        out_specs=pl.BlockSpec((n, nhid), lambda: (0, 0)),
        out_shape=jax.ShapeDtypeStruct((n, nhid), jnp.bfloat16),
    )(x, W1)

    body = functools.partial(_fused_kernel, blk=blk, nb=nb, cache=cache)
    out = pl.pallas_call(
        body,
        grid=(2, nb),
        in_specs=[
            pl.BlockSpec((n, nhid), lambda p, i: (0, 0)),
            pl.BlockSpec(memory_space=pl.ANY),
            pl.BlockSpec((nhid, nclass), lambda p, i: (0, 0)),
            pl.BlockSpec((1, nhid), lambda p, i: (0, 0)),
            pl.BlockSpec((1, nclass), lambda p, i: (0, 0)),
        ],
        out_specs=pl.BlockSpec(
            (blk, nclass),
            lambda p, i: (jnp.where(p == 0, i, (i + cache) % nb), 0)),
        out_shape=jax.ShapeDtypeStruct((n, nclass), jnp.float32),
        scratch_shapes=[
            pltpu.VMEM((2, blk, n), jnp.float32),
            pltpu.VMEM((max(cache, 1), blk, n), jnp.float8_e4m3fn),
            pltpu.VMEM((n, nclass), jnp.bfloat16),
            pltpu.SemaphoreType.DMA((2,)),
        ],
        compiler_params=pltpu.CompilerParams(
            dimension_semantics=("arbitrary", "arbitrary"),
        ),
    )(s1, adj, W4, b1_2d, b4_2d)
    return out
